# baseline (speedup 1.0000x reference)
import functools

import jax
import jax.numpy as jnp
from jax import lax
from jax.experimental import pallas as pl
from jax.experimental.pallas import tpu as pltpu
from jax.experimental.pallas import tpu_sc as plsc

NC, NS = 2, 16
NW = NC * NS
G = 32


def _precompute_body(h_ref, r_ref, ws_ref, wr_ref, kg_ref, wkg_ref, wkgb_ref,
                     aws_ref, awr_ref, combo_ref):
    d = h_ref.shape[1]
    a_dim = ws_ref.shape[1]
    pad = jnp.zeros((h_ref.shape[0], d - a_dim), jnp.float32)
    aws_ref[...] = jnp.concatenate(
        [jnp.dot(h_ref[...], ws_ref[...], preferred_element_type=jnp.float32),
         pad], axis=1)
    awr_ref[...] = jnp.concatenate(
        [jnp.dot(r_ref[...], wr_ref[...], preferred_element_type=jnp.float32),
         pad], axis=1)
    t = jnp.dot(kg_ref[...], wkg_ref[:d, :],
                preferred_element_type=jnp.float32)
    b2 = jnp.dot(kg_ref[...], wkg_ref[d:, :],
                 preferred_element_type=jnp.float32)
    combo_ref[...] = jnp.concatenate(
        [t[0:1] + b2[0:1], t[0:1] + b2[1:2], t[1:2] + b2[0:1], t[1:2] + b2[1:2]],
        axis=0) + wkgb_ref[...]


def _final_body(p_ref, wh_ref, out_ref):
    out_ref[...] = jnp.dot(jnp.sum(p_ref[...], axis=0), wh_ref[...],
                           preferred_element_type=jnp.float32)


def _sc_edge_kernel(acc_rows, d, a_dim, ng):
    rows_per_sub = acc_rows // NS
    ngp = ng + 2

    mesh = plsc.VectorSubcoreMesh(core_axis_name="c", subcore_axis_name="s",
                                  num_cores=NC)

    @functools.partial(
        pl.kernel,
        out_type=pltpu.HBM((NC, acc_rows, d), jnp.float32),
        mesh=mesh,
        compiler_params=pltpu.CompilerParams(needs_layout_passes=False),
        scratch_types=[
            [pltpu.VMEM((G, 8), jnp.int32)] * 2,
            [pltpu.VMEM((G,), jnp.int32)] * 2,
            [pltpu.VMEM((G,), jnp.int32)] * 2,
            [pltpu.VMEM((G,), jnp.int32)] * 2,
            [pltpu.VMEM((G,), jnp.int32)] * 2,
            [pltpu.VMEM((G, d), jnp.float32)] * 2,
            [pltpu.VMEM((G, d), jnp.float32)] * 2,
            [pltpu.VMEM((G, d), jnp.float32)] * 2,
            [pltpu.VMEM((G, d), jnp.float32)] * 2,
            pltpu.VMEM((G,), jnp.float32),
            pltpu.VMEM((4, a_dim), jnp.float32),
            pltpu.VMEM((a_dim, 16), jnp.float32),
            pltpu.VMEM((16,), jnp.float32),
            pltpu.VMEM((16,), jnp.int32),
            pltpu.VMEM_SHARED((acc_rows, d), jnp.float32),
            pltpu.SemaphoreType.DMA,
            pltpu.SemaphoreType.DMA,
        ],
    )
    def sc_kernel(e8_hbm, hid_hbm, rela_hbm, aws_hbm, awr_hbm, combo_hbm,
                  w_hbm, wb_hbm, lf_hbm, out_hbm,
                  ev, sub_v, rel_v, obj_v, c_v, hs_v, hr_v, as_v, ar_v,
                  alpha_v, combo_v, w_v, wb_v, lf_v,
                  accum, sem, sem_e):
        c = lax.axis_index("c")
        s = lax.axis_index("s")
        wid = s * NC + c

        pltpu.sync_copy(combo_hbm, combo_v)
        pltpu.sync_copy(w_hbm, w_v)
        pltpu.sync_copy(wb_hbm, wb_v)
        pltpu.sync_copy(lf_hbm, lf_v)

        def zrow(r, carry):
            for k in range(d // 16):
                hs_v[0][r, pl.ds(16 * k, 16)] = jnp.zeros((16,), jnp.float32)
            return carry
        lax.fori_loop(0, G, zrow, 0)
        for j in range(rows_per_sub // G):
            pltpu.sync_copy(hs_v[0],
                            accum.at[pl.ds(s * rows_per_sub + j * G, G)])
        rem = rows_per_sub % G
        if rem:
            pltpu.sync_copy(
                hs_v[0].at[pl.ds(0, rem)],
                accum.at[pl.ds(s * rows_per_sub + rows_per_sub - rem, rem)])
        plsc.subcore_barrier()

        left_vec = lf_v[...]
        wb_vec = wb_v[...]

        def ebase(g):
            return (g * NW + wid) * G

        def extract_and_fire(h):
            for q in range(G // 16):
                evec = lax.iota(jnp.int32, 16) + q * 16
                head = plsc.load_gather(ev[h], [evec, jnp.full((16,), 1, jnp.int32)])
                rel = plsc.load_gather(ev[h], [evec, jnp.full((16,), 2, jnp.int32)])
                tail = plsc.load_gather(ev[h], [evec, jnp.full((16,), 3, jnp.int32)])
                sub = plsc.load_gather(ev[h], [evec, jnp.full((16,), 4, jnp.int32)])
                obj = plsc.load_gather(ev[h], [evec, jnp.full((16,), 5, jnp.int32)])
                cvec = (2 * (head >= left_vec).astype(jnp.int32)
                        + (tail >= left_vec).astype(jnp.int32))
                sub_v[h][pl.ds(q * 16, 16)] = sub
                rel_v[h][pl.ds(q * 16, 16)] = rel
                obj_v[h][pl.ds(q * 16, 16)] = obj
                c_v[h][pl.ds(q * 16, 16)] = cvec
            pltpu.async_copy(hid_hbm.at[sub_v[h]], hs_v[h], sem)
            pltpu.async_copy(rela_hbm.at[rel_v[h]], hr_v[h], sem)
            pltpu.async_copy(aws_hbm.at[sub_v[h]], as_v[h], sem)
            pltpu.async_copy(awr_hbm.at[rel_v[h]], ar_v[h], sem)

        def drain_gathers(h):
            pltpu.make_async_copy(hid_hbm.at[sub_v[h]], hs_v[h], sem).wait()
            pltpu.make_async_copy(rela_hbm.at[rel_v[h]], hr_v[h], sem).wait()
            pltpu.make_async_copy(aws_hbm.at[sub_v[h]], as_v[h], sem).wait()
            pltpu.make_async_copy(awr_hbm.at[rel_v[h]], ar_v[h], sem).wait()

        asum_base = a_dim

        def compute(h):
            @plsc.parallel_loop(0, G, unroll=8)
            def asum_row(e):
                csp = plsc.load_gather(c_v[h], [jnp.full((16,), 0, jnp.int32) + e])
                for kq in range(a_dim // 16):
                    cb = plsc.load_gather(
                        combo_v, [csp, lax.iota(jnp.int32, 16) + kq * 16])
                    as_v[h][e, pl.ds(asum_base + kq * 16, 16)] = (
                        as_v[h][e, pl.ds(kq * 16, 16)]
                        + ar_v[h][e, pl.ds(kq * 16, 16)] + cb)

            for q in range(G // 16):
                evec = lax.iota(jnp.int32, 16) + q * 16
                acc = jnp.zeros((16,), jnp.float32)
                for k in range(a_dim):
                    av = plsc.load_gather(
                        as_v[h], [evec, jnp.full((16,), asum_base + k, jnp.int32)])
                    acc = acc + jnp.maximum(av, 0.0) * w_v[k, :]
                x = acc + wb_vec
                alpha = 1.0 / (1.0 + jnp.exp(-x))
                plsc.store_scatter(alpha_v, [evec], alpha)

            @plsc.parallel_loop(0, G, unroll=8)
            def msg_row(e):
                a_e = plsc.load_gather(alpha_v, [jnp.full((16,), 0, jnp.int32) + e])
                for sl in range(d // 16):
                    hr_v[h][e, pl.ds(16 * sl, 16)] = a_e * (
                        hs_v[h][e, pl.ds(16 * sl, 16)]
                        + hr_v[h][e, pl.ds(16 * sl, 16)])

            pltpu.sync_copy(hr_v[h], accum.at[obj_v[h]], add=True)

        pltpu.sync_copy(e8_hbm.at[pl.ds(ebase(0), G)], ev[0])
        pltpu.async_copy(e8_hbm.at[pl.ds(ebase(1), G)], ev[1], sem_e)
        extract_and_fire(0)

        def pair(i, carry):
            for h in range(2):
                g = 2 * i + h
                o = 1 - h
                pltpu.make_async_copy(
                    e8_hbm.at[pl.ds(ebase(g + 1), G)], ev[o], sem_e).wait()
                extract_and_fire(o)
                pltpu.async_copy(
                    e8_hbm.at[pl.ds(ebase(g + 2), G)], ev[h], sem_e)
                drain_gathers(h)
                compute(h)
            return carry
        lax.fori_loop(0, ng // 2, pair, 0)

        pltpu.make_async_copy(
            e8_hbm.at[pl.ds(ebase(ng + 1), G)], ev[1], sem_e).wait()
        drain_gathers(0)

        plsc.subcore_barrier()
        for j in range(rows_per_sub // G):
            r0 = s * rows_per_sub + j * G
            pltpu.sync_copy(accum.at[pl.ds(r0, G)], out_hbm.at[c, pl.ds(r0, G)])
        if rows_per_sub % G:
            rem2 = rows_per_sub % G
            r0 = s * rows_per_sub + rows_per_sub - rem2
            pltpu.sync_copy(accum.at[pl.ds(r0, rem2)],
                            out_hbm.at[c, pl.ds(r0, rem2)])

    return sc_kernel


def kernel(hidden, edges, n_node, kgemb, left_num, rela_embed, Ws_attn,
           Wr_attn, Wkg_attn_W, Wkg_attn_b, w_alpha_W, w_alpha_b, W_h):
    n, d = hidden.shape
    a_dim = Ws_attn.shape[1]
    e = edges.shape[0]

    ng = -(-e // (NW * G))
    ng = ng + (ng % 2)
    e_pad = NW * ng * G
    acc_rows = -(-(n + 1) // (NS * 8)) * (NS * 8)
    blk = 1024
    n_pad = -(-max(n, rela_embed.shape[0]) // blk) * blk

    hid_pad = jnp.pad(hidden, ((0, n_pad - n), (0, 0)))
    rela_pad = jnp.pad(rela_embed, ((0, n_pad - rela_embed.shape[0]), (0, 0)))
    e6 = jnp.pad(edges.astype(jnp.int32), ((0, 0), (0, 2)))
    dummy = jnp.broadcast_to(
        jnp.array([0, 0, 0, 0, 0, n, 0, 0], jnp.int32),
        (NW * (ng + 2) * G - e, 8))
    e8 = jnp.concatenate([e6, dummy], axis=0)
    w_bc = jnp.broadcast_to(
        w_alpha_W.reshape((a_dim, 1)).astype(jnp.float32), (a_dim, 16))
    wb16 = jnp.broadcast_to(w_alpha_b.astype(jnp.float32), (16,))
    lf16 = jnp.broadcast_to(jnp.asarray(left_num, jnp.int32), (16,))
    wkgb = Wkg_attn_b.reshape((1, a_dim)).astype(jnp.float32)

    grid = n_pad // blk
    aws, awr, combo4 = pl.pallas_call(
        _precompute_body,
        grid=(grid,),
        in_specs=[
            pl.BlockSpec((blk, d), lambda i: (i, 0)),
            pl.BlockSpec((blk, d), lambda i: (i, 0)),
            pl.BlockSpec((d, a_dim), lambda i: (0, 0)),
            pl.BlockSpec((d, a_dim), lambda i: (0, 0)),
            pl.BlockSpec((2, d), lambda i: (0, 0)),
            pl.BlockSpec((2 * d, a_dim), lambda i: (0, 0)),
            pl.BlockSpec((1, a_dim), lambda i: (0, 0)),
        ],
        out_specs=[
            pl.BlockSpec((blk, d), lambda i: (i, 0)),
            pl.BlockSpec((blk, d), lambda i: (i, 0)),
            pl.BlockSpec((4, a_dim), lambda i: (0, 0)),
        ],
        out_shape=[
            jax.ShapeDtypeStruct((n_pad, d), jnp.float32),
            jax.ShapeDtypeStruct((n_pad, d), jnp.float32),
            jax.ShapeDtypeStruct((4, a_dim), jnp.float32),
        ],
    )(hid_pad, rela_pad, Ws_attn, Wr_attn, kgemb, Wkg_attn_W, wkgb)

    parts = _sc_edge_kernel(acc_rows, d, a_dim, ng)(
        e8, hidden, rela_embed, aws, awr, combo4, w_bc, wb16, lf16)

    if n % 1000 == 0:
        blk2, grid2 = 1000, n // 1000
    else:
        blk2, grid2 = acc_rows // NS, NS
    out_pad = pl.pallas_call(
        _final_body,
        grid=(grid2,),
        in_specs=[
            pl.BlockSpec((NC, blk2, d), lambda i: (0, i, 0)),
            pl.BlockSpec((d, d), lambda i: (0, 0)),
        ],
        out_specs=pl.BlockSpec((blk2, d), lambda i: (i, 0)),
        out_shape=jax.ShapeDtypeStruct((blk2 * grid2, d), jnp.float32),
    )(parts, W_h)

    return out_pad[:n]


# speedup vs baseline: 1.1478x; 1.1478x over previous
import functools

import jax
import jax.numpy as jnp
from jax import lax
from jax.experimental import pallas as pl
from jax.experimental.pallas import tpu as pltpu
from jax.experimental.pallas import tpu_sc as plsc

NC, NS = 2, 16
NW = NC * NS
G = 32


def _precompute_body(h_ref, r_ref, ws_ref, wr_ref, kg_ref, wkg_ref, wkgb_ref,
                     aws_ref, awr_ref, combo_ref):
    d = h_ref.shape[1]
    a_dim = ws_ref.shape[1]
    pad = jnp.zeros((h_ref.shape[0], d - a_dim), jnp.float32)
    aws_ref[...] = jnp.concatenate(
        [jnp.dot(h_ref[...], ws_ref[...], preferred_element_type=jnp.float32),
         pad], axis=1)
    awr_ref[...] = jnp.concatenate(
        [jnp.dot(r_ref[...], wr_ref[...], preferred_element_type=jnp.float32),
         pad], axis=1)
    t = jnp.dot(kg_ref[...], wkg_ref[:d, :],
                preferred_element_type=jnp.float32)
    b2 = jnp.dot(kg_ref[...], wkg_ref[d:, :],
                 preferred_element_type=jnp.float32)
    combo_ref[...] = jnp.concatenate(
        [t[0:1] + b2[0:1], t[0:1] + b2[1:2], t[1:2] + b2[0:1], t[1:2] + b2[1:2]],
        axis=0) + wkgb_ref[...]


def _final_body(p_ref, wh_ref, out_ref):
    out_ref[...] = jnp.dot(jnp.sum(p_ref[...], axis=0), wh_ref[...],
                           preferred_element_type=jnp.float32)


def _sc_edge_kernel(acc_rows, d, a_dim, ng):
    rows_per_sub = acc_rows // NS
    ngp = ng + 2

    mesh = plsc.VectorSubcoreMesh(core_axis_name="c", subcore_axis_name="s",
                                  num_cores=NC)

    @functools.partial(
        pl.kernel,
        out_type=pltpu.HBM((NC, acc_rows, d), jnp.float32),
        mesh=mesh,
        compiler_params=pltpu.CompilerParams(needs_layout_passes=False),
        scratch_types=[
            [pltpu.VMEM((G, 8), jnp.int32)] * 2,
            [pltpu.VMEM((G,), jnp.int32)] * 2,
            [pltpu.VMEM((G,), jnp.int32)] * 2,
            [pltpu.VMEM((G,), jnp.int32)] * 2,
            [pltpu.VMEM((G,), jnp.int32)] * 2,
            [pltpu.VMEM((G, d), jnp.float32)] * 2,
            [pltpu.VMEM((G, d), jnp.float32)] * 2,
            [pltpu.VMEM((G, d), jnp.float32)] * 2,
            [pltpu.VMEM((G, d), jnp.float32)] * 2,
            pltpu.VMEM((G,), jnp.float32),
            pltpu.VMEM((4, a_dim), jnp.float32),
            pltpu.VMEM((a_dim, 16), jnp.float32),
            pltpu.VMEM((16,), jnp.float32),
            pltpu.VMEM((16,), jnp.int32),
            pltpu.VMEM_SHARED((acc_rows, d), jnp.float32),
            pltpu.SemaphoreType.DMA,
            pltpu.SemaphoreType.DMA,
        ],
    )
    def sc_kernel(e8_hbm, hid_hbm, rela_hbm, aws_hbm, awr_hbm, combo_hbm,
                  w_hbm, wb_hbm, lf_hbm, out_hbm,
                  ev, sub_v, rel_v, obj_v, c_v, hs_v, hr_v, as_v, ar_v,
                  alpha_v, combo_v, w_v, wb_v, lf_v,
                  accum, sem, sem_e):
        c = lax.axis_index("c")
        s = lax.axis_index("s")
        wid = s * NC + c

        pltpu.sync_copy(combo_hbm, combo_v)
        pltpu.sync_copy(w_hbm, w_v)
        pltpu.sync_copy(wb_hbm, wb_v)
        pltpu.sync_copy(lf_hbm, lf_v)

        def zrow(r, carry):
            for k in range(d // 16):
                hs_v[0][r, pl.ds(16 * k, 16)] = jnp.zeros((16,), jnp.float32)
            return carry
        lax.fori_loop(0, G, zrow, 0)
        for j in range(rows_per_sub // G):
            pltpu.sync_copy(hs_v[0],
                            accum.at[pl.ds(s * rows_per_sub + j * G, G)])
        rem = rows_per_sub % G
        if rem:
            pltpu.sync_copy(
                hs_v[0].at[pl.ds(0, rem)],
                accum.at[pl.ds(s * rows_per_sub + rows_per_sub - rem, rem)])
        plsc.subcore_barrier()

        left_vec = lf_v[...]
        wb_vec = wb_v[...]

        def ebase(g):
            return (g * NW + wid) * G

        def extract_and_fire(h):
            for q in range(G // 16):
                evec = lax.iota(jnp.int32, 16) + q * 16
                head = plsc.load_gather(ev[h], [evec, jnp.full((16,), 1, jnp.int32)])
                rel = plsc.load_gather(ev[h], [evec, jnp.full((16,), 2, jnp.int32)])
                tail = plsc.load_gather(ev[h], [evec, jnp.full((16,), 3, jnp.int32)])
                sub = plsc.load_gather(ev[h], [evec, jnp.full((16,), 4, jnp.int32)])
                obj = plsc.load_gather(ev[h], [evec, jnp.full((16,), 5, jnp.int32)])
                cvec = (2 * (head >= left_vec).astype(jnp.int32)
                        + (tail >= left_vec).astype(jnp.int32))
                sub_v[h][pl.ds(q * 16, 16)] = sub
                rel_v[h][pl.ds(q * 16, 16)] = rel
                obj_v[h][pl.ds(q * 16, 16)] = obj
                c_v[h][pl.ds(q * 16, 16)] = cvec
            pltpu.async_copy(hid_hbm.at[sub_v[h]], hs_v[h], sem)
            pltpu.async_copy(rela_hbm.at[rel_v[h]], hr_v[h], sem)
            pltpu.async_copy(aws_hbm.at[sub_v[h]], as_v[h], sem)
            pltpu.async_copy(awr_hbm.at[rel_v[h]], ar_v[h], sem)

        def drain_gathers(h):
            pltpu.make_async_copy(hid_hbm.at[sub_v[h]], hs_v[h], sem).wait()
            pltpu.make_async_copy(rela_hbm.at[rel_v[h]], hr_v[h], sem).wait()
            pltpu.make_async_copy(aws_hbm.at[sub_v[h]], as_v[h], sem).wait()
            pltpu.make_async_copy(awr_hbm.at[rel_v[h]], ar_v[h], sem).wait()

        asum_base = a_dim

        def compute(h):
            @plsc.parallel_loop(0, G, unroll=4)
            def asum_row(e):
                csp = plsc.load_gather(c_v[h], [jnp.full((16,), 0, jnp.int32) + e])
                for kq in range(a_dim // 16):
                    cb = plsc.load_gather(
                        combo_v, [csp, lax.iota(jnp.int32, 16) + kq * 16])
                    as_v[h][e, pl.ds(asum_base + kq * 16, 16)] = (
                        as_v[h][e, pl.ds(kq * 16, 16)]
                        + ar_v[h][e, pl.ds(kq * 16, 16)] + cb)

            for q in range(G // 16):
                evec = lax.iota(jnp.int32, 16) + q * 16
                acc = jnp.zeros((16,), jnp.float32)
                for k in range(a_dim):
                    av = plsc.load_gather(
                        as_v[h], [evec, jnp.full((16,), asum_base + k, jnp.int32)])
                    acc = acc + jnp.maximum(av, 0.0) * w_v[k, :]
                x = acc + wb_vec
                alpha = 1.0 / (1.0 + jnp.exp(-x))
                plsc.store_scatter(alpha_v, [evec], alpha)

            @plsc.parallel_loop(0, G, unroll=4)
            def msg_row(e):
                a_e = plsc.load_gather(alpha_v, [jnp.full((16,), 0, jnp.int32) + e])
                for sl in range(d // 16):
                    hr_v[h][e, pl.ds(16 * sl, 16)] = a_e * (
                        hs_v[h][e, pl.ds(16 * sl, 16)]
                        + hr_v[h][e, pl.ds(16 * sl, 16)])

            pltpu.sync_copy(hr_v[h], accum.at[obj_v[h]], add=True)

        pltpu.sync_copy(e8_hbm.at[pl.ds(ebase(0), G)], ev[0])
        pltpu.async_copy(e8_hbm.at[pl.ds(ebase(1), G)], ev[1], sem_e)
        extract_and_fire(0)

        def pair(i, carry):
            for h in range(2):
                g = 2 * i + h
                o = 1 - h
                pltpu.make_async_copy(
                    e8_hbm.at[pl.ds(ebase(g + 1), G)], ev[o], sem_e).wait()
                extract_and_fire(o)
                pltpu.async_copy(
                    e8_hbm.at[pl.ds(ebase(g + 2), G)], ev[h], sem_e)
                drain_gathers(h)
                compute(h)
            return carry
        lax.fori_loop(0, ng // 2, pair, 0)

        pltpu.make_async_copy(
            e8_hbm.at[pl.ds(ebase(ng + 1), G)], ev[1], sem_e).wait()
        drain_gathers(0)

        plsc.subcore_barrier()
        for j in range(rows_per_sub // G):
            r0 = s * rows_per_sub + j * G
            pltpu.sync_copy(accum.at[pl.ds(r0, G)], out_hbm.at[c, pl.ds(r0, G)])
        if rows_per_sub % G:
            rem2 = rows_per_sub % G
            r0 = s * rows_per_sub + rows_per_sub - rem2
            pltpu.sync_copy(accum.at[pl.ds(r0, rem2)],
                            out_hbm.at[c, pl.ds(r0, rem2)])

    return sc_kernel


def kernel(hidden, edges, n_node, kgemb, left_num, rela_embed, Ws_attn,
           Wr_attn, Wkg_attn_W, Wkg_attn_b, w_alpha_W, w_alpha_b, W_h):
    n, d = hidden.shape
    a_dim = Ws_attn.shape[1]
    e = edges.shape[0]

    ng = -(-e // (NW * G))
    ng = ng + (ng % 2)
    e_pad = NW * ng * G
    acc_rows = -(-(n + 1) // (NS * 8)) * (NS * 8)
    blk = 1024
    n_pad = -(-max(n, rela_embed.shape[0]) // blk) * blk

    hid_pad = jnp.pad(hidden, ((0, n_pad - n), (0, 0)))
    rela_pad = jnp.pad(rela_embed, ((0, n_pad - rela_embed.shape[0]), (0, 0)))
    e6 = jnp.pad(edges.astype(jnp.int32), ((0, 0), (0, 2)))
    dummy = jnp.broadcast_to(
        jnp.array([0, 0, 0, 0, 0, n, 0, 0], jnp.int32),
        (NW * (ng + 2) * G - e, 8))
    e8 = jnp.concatenate([e6, dummy], axis=0)
    w_bc = jnp.broadcast_to(
        w_alpha_W.reshape((a_dim, 1)).astype(jnp.float32), (a_dim, 16))
    wb16 = jnp.broadcast_to(w_alpha_b.astype(jnp.float32), (16,))
    lf16 = jnp.broadcast_to(jnp.asarray(left_num, jnp.int32), (16,))
    wkgb = Wkg_attn_b.reshape((1, a_dim)).astype(jnp.float32)

    grid = n_pad // blk
    aws, awr, combo4 = pl.pallas_call(
        _precompute_body,
        grid=(grid,),
        in_specs=[
            pl.BlockSpec((blk, d), lambda i: (i, 0)),
            pl.BlockSpec((blk, d), lambda i: (i, 0)),
            pl.BlockSpec((d, a_dim), lambda i: (0, 0)),
            pl.BlockSpec((d, a_dim), lambda i: (0, 0)),
            pl.BlockSpec((2, d), lambda i: (0, 0)),
            pl.BlockSpec((2 * d, a_dim), lambda i: (0, 0)),
            pl.BlockSpec((1, a_dim), lambda i: (0, 0)),
        ],
        out_specs=[
            pl.BlockSpec((blk, d), lambda i: (i, 0)),
            pl.BlockSpec((blk, d), lambda i: (i, 0)),
            pl.BlockSpec((4, a_dim), lambda i: (0, 0)),
        ],
        out_shape=[
            jax.ShapeDtypeStruct((n_pad, d), jnp.float32),
            jax.ShapeDtypeStruct((n_pad, d), jnp.float32),
            jax.ShapeDtypeStruct((4, a_dim), jnp.float32),
        ],
    )(hid_pad, rela_pad, Ws_attn, Wr_attn, kgemb, Wkg_attn_W, wkgb)

    parts = _sc_edge_kernel(acc_rows, d, a_dim, ng)(
        e8, hidden, rela_embed, aws, awr, combo4, w_bc, wb16, lf16)

    if n % 1000 == 0:
        blk2, grid2 = 1000, n // 1000
    else:
        blk2, grid2 = acc_rows // NS, NS
    out_pad = pl.pallas_call(
        _final_body,
        grid=(grid2,),
        in_specs=[
            pl.BlockSpec((NC, blk2, d), lambda i: (0, i, 0)),
            pl.BlockSpec((d, d), lambda i: (0, 0)),
        ],
        out_specs=pl.BlockSpec((blk2, d), lambda i: (i, 0)),
        out_shape=jax.ShapeDtypeStruct((blk2 * grid2, d), jnp.float32),
    )(parts, W_h)

    return out_pad[:n]


# speedup vs baseline: 1.1479x; 1.0001x over previous
import functools

import jax
import jax.numpy as jnp
from jax import lax
from jax.experimental import pallas as pl
from jax.experimental.pallas import tpu as pltpu
from jax.experimental.pallas import tpu_sc as plsc

NC, NS = 2, 16
NW = NC * NS
G = 32


def _precompute_body(h_ref, r_ref, ws_ref, wr_ref, kg_ref, wkg_ref, wkgb_ref,
                     aws_ref, awr_ref, combo_ref):
    d = h_ref.shape[1]
    a_dim = ws_ref.shape[1]
    pad = jnp.zeros((h_ref.shape[0], d - a_dim), jnp.float32)
    aws_ref[...] = jnp.concatenate(
        [jnp.dot(h_ref[...], ws_ref[...], preferred_element_type=jnp.float32),
         pad], axis=1)
    awr_ref[...] = jnp.concatenate(
        [jnp.dot(r_ref[...], wr_ref[...], preferred_element_type=jnp.float32),
         pad], axis=1)
    t = jnp.dot(kg_ref[...], wkg_ref[:d, :],
                preferred_element_type=jnp.float32)
    b2 = jnp.dot(kg_ref[...], wkg_ref[d:, :],
                 preferred_element_type=jnp.float32)
    combo_ref[...] = jnp.concatenate(
        [t[0:1] + b2[0:1], t[0:1] + b2[1:2], t[1:2] + b2[0:1], t[1:2] + b2[1:2]],
        axis=0) + wkgb_ref[...]


def _final_body(p_ref, wh_ref, out_ref):
    out_ref[...] = jnp.dot(jnp.sum(p_ref[...], axis=0), wh_ref[...],
                           preferred_element_type=jnp.float32)


def _sc_edge_kernel(acc_rows, d, a_dim, ng):
    rows_per_sub = acc_rows // NS
    ngp = ng + 2

    mesh = plsc.VectorSubcoreMesh(core_axis_name="c", subcore_axis_name="s",
                                  num_cores=NC)

    @functools.partial(
        pl.kernel,
        out_type=pltpu.HBM((NC, acc_rows, d), jnp.float32),
        mesh=mesh,
        compiler_params=pltpu.CompilerParams(needs_layout_passes=False),
        scratch_types=[
            [pltpu.VMEM((G, 8), jnp.int32)] * 2,
            [pltpu.VMEM((G,), jnp.int32)] * 2,
            [pltpu.VMEM((G,), jnp.int32)] * 2,
            [pltpu.VMEM((G,), jnp.int32)] * 2,
            [pltpu.VMEM((G,), jnp.int32)] * 2,
            [pltpu.VMEM((G, d), jnp.float32)] * 2,
            [pltpu.VMEM((G, d), jnp.float32)] * 2,
            [pltpu.VMEM((G, d), jnp.float32)] * 2,
            [pltpu.VMEM((G, d), jnp.float32)] * 2,
            pltpu.VMEM((G,), jnp.float32),
            pltpu.VMEM((4, a_dim), jnp.float32),
            pltpu.VMEM((a_dim, 16), jnp.float32),
            pltpu.VMEM((16,), jnp.float32),
            pltpu.VMEM((16,), jnp.int32),
            pltpu.VMEM_SHARED((acc_rows, d), jnp.float32),
            pltpu.SemaphoreType.DMA,
            pltpu.SemaphoreType.DMA,
        ],
    )
    def sc_kernel(e8_hbm, hid_hbm, rela_hbm, aws_hbm, awr_hbm, combo_hbm,
                  w_hbm, wb_hbm, lf_hbm, out_hbm,
                  ev, sub_v, rel_v, obj_v, c_v, hs_v, hr_v, as_v, ar_v,
                  alpha_v, combo_v, w_v, wb_v, lf_v,
                  accum, sem, sem_e):
        c = lax.axis_index("c")
        s = lax.axis_index("s")
        wid = s * NC + c

        pltpu.sync_copy(combo_hbm, combo_v)
        pltpu.sync_copy(w_hbm, w_v)
        pltpu.sync_copy(wb_hbm, wb_v)
        pltpu.sync_copy(lf_hbm, lf_v)

        def zrow(r, carry):
            for k in range(d // 16):
                hs_v[0][r, pl.ds(16 * k, 16)] = jnp.zeros((16,), jnp.float32)
            return carry
        lax.fori_loop(0, G, zrow, 0)
        for j in range(rows_per_sub // G):
            pltpu.sync_copy(hs_v[0],
                            accum.at[pl.ds(s * rows_per_sub + j * G, G)])
        rem = rows_per_sub % G
        if rem:
            pltpu.sync_copy(
                hs_v[0].at[pl.ds(0, rem)],
                accum.at[pl.ds(s * rows_per_sub + rows_per_sub - rem, rem)])
        plsc.subcore_barrier()

        left_vec = lf_v[...]
        wb_vec = wb_v[...]

        def ebase(g):
            return (g * NW + wid) * G

        def extract_and_fire(h):
            for q in range(G // 16):
                evec = lax.iota(jnp.int32, 16) + q * 16
                head = plsc.load_gather(ev[h], [evec, jnp.full((16,), 1, jnp.int32)])
                rel = plsc.load_gather(ev[h], [evec, jnp.full((16,), 2, jnp.int32)])
                tail = plsc.load_gather(ev[h], [evec, jnp.full((16,), 3, jnp.int32)])
                sub = plsc.load_gather(ev[h], [evec, jnp.full((16,), 4, jnp.int32)])
                obj = plsc.load_gather(ev[h], [evec, jnp.full((16,), 5, jnp.int32)])
                cvec = (2 * (head >= left_vec).astype(jnp.int32)
                        + (tail >= left_vec).astype(jnp.int32))
                sub_v[h][pl.ds(q * 16, 16)] = sub
                rel_v[h][pl.ds(q * 16, 16)] = rel
                obj_v[h][pl.ds(q * 16, 16)] = obj
                c_v[h][pl.ds(q * 16, 16)] = cvec
            pltpu.async_copy(hid_hbm.at[sub_v[h]], hs_v[h], sem)
            pltpu.async_copy(rela_hbm.at[rel_v[h]], hr_v[h], sem)
            pltpu.async_copy(aws_hbm.at[sub_v[h]], as_v[h], sem)
            pltpu.async_copy(awr_hbm.at[rel_v[h]], ar_v[h], sem)

        def drain_gathers(h):
            pltpu.make_async_copy(hid_hbm.at[sub_v[h]], hs_v[h], sem).wait()
            pltpu.make_async_copy(rela_hbm.at[rel_v[h]], hr_v[h], sem).wait()
            pltpu.make_async_copy(aws_hbm.at[sub_v[h]], as_v[h], sem).wait()
            pltpu.make_async_copy(awr_hbm.at[rel_v[h]], ar_v[h], sem).wait()

        asum_base = a_dim

        def compute(h):
            @plsc.parallel_loop(0, G, unroll=4)
            def asum_row(e):
                csp = plsc.load_gather(c_v[h], [jnp.full((16,), 0, jnp.int32) + e])
                for kq in range(a_dim // 16):
                    cb = plsc.load_gather(
                        combo_v, [csp, lax.iota(jnp.int32, 16) + kq * 16])
                    as_v[h][e, pl.ds(asum_base + kq * 16, 16)] = (
                        as_v[h][e, pl.ds(kq * 16, 16)]
                        + ar_v[h][e, pl.ds(kq * 16, 16)] + cb)

            for q in range(G // 16):
                evec = lax.iota(jnp.int32, 16) + q * 16
                acc = jnp.zeros((16,), jnp.float32)
                for k in range(a_dim):
                    av = plsc.load_gather(
                        as_v[h], [evec, jnp.full((16,), asum_base + k, jnp.int32)])
                    acc = acc + jnp.maximum(av, 0.0) * w_v[k, :]
                x = acc + wb_vec
                alpha = 1.0 / (1.0 + jnp.exp(-x))
                plsc.store_scatter(alpha_v, [evec], alpha)

            @plsc.parallel_loop(0, G, unroll=4)
            def msg_row(e):
                a_e = plsc.load_gather(alpha_v, [jnp.full((16,), 0, jnp.int32) + e])
                for sl in range(d // 16):
                    hr_v[h][e, pl.ds(16 * sl, 16)] = a_e * (
                        hs_v[h][e, pl.ds(16 * sl, 16)]
                        + hr_v[h][e, pl.ds(16 * sl, 16)])

            pltpu.sync_copy(hr_v[h], accum.at[obj_v[h]], add=True)

        pltpu.sync_copy(e8_hbm.at[pl.ds(ebase(0), G)], ev[0])
        pltpu.async_copy(e8_hbm.at[pl.ds(ebase(1), G)], ev[1], sem_e)
        extract_and_fire(0)

        def pair(i, carry):
            for h in range(2):
                g = 2 * i + h
                o = 1 - h
                pltpu.make_async_copy(
                    e8_hbm.at[pl.ds(ebase(g + 1), G)], ev[o], sem_e).wait()
                extract_and_fire(o)
                pltpu.async_copy(
                    e8_hbm.at[pl.ds(ebase(g + 2), G)], ev[h], sem_e)
                drain_gathers(h)
                compute(h)
            return carry
        lax.fori_loop(0, ng // 2, pair, 0)

        pltpu.make_async_copy(
            e8_hbm.at[pl.ds(ebase(ng + 1), G)], ev[1], sem_e).wait()
        drain_gathers(0)

        plsc.subcore_barrier()
        for j in range(rows_per_sub // G):
            r0 = s * rows_per_sub + j * G
            pltpu.sync_copy(accum.at[pl.ds(r0, G)], out_hbm.at[c, pl.ds(r0, G)])
        if rows_per_sub % G:
            rem2 = rows_per_sub % G
            r0 = s * rows_per_sub + rows_per_sub - rem2
            pltpu.sync_copy(accum.at[pl.ds(r0, rem2)],
                            out_hbm.at[c, pl.ds(r0, rem2)])

    return sc_kernel


def kernel(hidden, edges, n_node, kgemb, left_num, rela_embed, Ws_attn,
           Wr_attn, Wkg_attn_W, Wkg_attn_b, w_alpha_W, w_alpha_b, W_h):
    n, d = hidden.shape
    a_dim = Ws_attn.shape[1]
    e = edges.shape[0]

    ng = -(-e // (NW * G))
    ng = ng + (ng % 2)
    acc_rows = -(-(n + 1) // (NS * 8)) * (NS * 8)
    blk = 1000 if n % 1000 == 0 else 8 * (n // 8)
    grid = n // blk

    e6 = jnp.pad(edges.astype(jnp.int32), ((0, 0), (0, 2)))
    dummy = jnp.broadcast_to(
        jnp.array([0, 0, 0, 0, 0, n, 0, 0], jnp.int32),
        (NW * (ng + 2) * G - e, 8))
    e8 = jnp.concatenate([e6, dummy], axis=0)
    w_bc = jnp.broadcast_to(
        w_alpha_W.reshape((a_dim, 1)).astype(jnp.float32), (a_dim, 16))
    wb16 = jnp.broadcast_to(w_alpha_b.astype(jnp.float32), (16,))
    lf16 = jnp.broadcast_to(jnp.asarray(left_num, jnp.int32), (16,))
    wkgb = Wkg_attn_b.reshape((1, a_dim)).astype(jnp.float32)

    aws, awr, combo4 = pl.pallas_call(
        _precompute_body,
        grid=(grid,),
        in_specs=[
            pl.BlockSpec((blk, d), lambda i: (i, 0)),
            pl.BlockSpec((blk, d), lambda i: (i, 0)),
            pl.BlockSpec((d, a_dim), lambda i: (0, 0)),
            pl.BlockSpec((d, a_dim), lambda i: (0, 0)),
            pl.BlockSpec((2, d), lambda i: (0, 0)),
            pl.BlockSpec((2 * d, a_dim), lambda i: (0, 0)),
            pl.BlockSpec((1, a_dim), lambda i: (0, 0)),
        ],
        out_specs=[
            pl.BlockSpec((blk, d), lambda i: (i, 0)),
            pl.BlockSpec((blk, d), lambda i: (i, 0)),
            pl.BlockSpec((4, a_dim), lambda i: (0, 0)),
        ],
        out_shape=[
            jax.ShapeDtypeStruct((n, d), jnp.float32),
            jax.ShapeDtypeStruct((n, d), jnp.float32),
            jax.ShapeDtypeStruct((4, a_dim), jnp.float32),
        ],
    )(hidden, rela_embed, Ws_attn, Wr_attn, kgemb, Wkg_attn_W, wkgb)

    parts = _sc_edge_kernel(acc_rows, d, a_dim, ng)(
        e8, hidden, rela_embed, aws, awr, combo4, w_bc, wb16, lf16)

    if n % 1000 == 0:
        blk2, grid2 = 1000, n // 1000
    else:
        blk2, grid2 = acc_rows // NS, NS
    out_pad = pl.pallas_call(
        _final_body,
        grid=(grid2,),
        in_specs=[
            pl.BlockSpec((NC, blk2, d), lambda i: (0, i, 0)),
            pl.BlockSpec((d, d), lambda i: (0, 0)),
        ],
        out_specs=pl.BlockSpec((blk2, d), lambda i: (i, 0)),
        out_shape=jax.ShapeDtypeStruct((blk2 * grid2, d), jnp.float32),
    )(parts, W_h)

    return out_pad[:n]


# speedup vs baseline: 1.1512x; 1.0029x over previous
import functools

import jax
import jax.numpy as jnp
from jax import lax
from jax.experimental import pallas as pl
from jax.experimental.pallas import tpu as pltpu
from jax.experimental.pallas import tpu_sc as plsc

NC, NS = 2, 16
NW = NC * NS
G = 32


def _precompute_body(h_ref, r_ref, ws_ref, wr_ref, kg_ref, wkg_ref, wkgb_ref,
                     aws_ref, awr_ref, combo_ref):
    d = h_ref.shape[1]
    a_dim = ws_ref.shape[1]
    pad = jnp.zeros((h_ref.shape[0], d - a_dim), jnp.float32)
    aws_ref[...] = jnp.concatenate(
        [jnp.dot(h_ref[...], ws_ref[...], preferred_element_type=jnp.float32),
         pad], axis=1)
    awr_ref[...] = jnp.concatenate(
        [jnp.dot(r_ref[...], wr_ref[...], preferred_element_type=jnp.float32),
         pad], axis=1)
    t = jnp.dot(kg_ref[...], wkg_ref[:d, :],
                preferred_element_type=jnp.float32)
    b2 = jnp.dot(kg_ref[...], wkg_ref[d:, :],
                 preferred_element_type=jnp.float32)
    combo_ref[...] = jnp.concatenate(
        [t[0:1] + b2[0:1], t[0:1] + b2[1:2], t[1:2] + b2[0:1], t[1:2] + b2[1:2]],
        axis=0) + wkgb_ref[...]


def _final_body(p_ref, wh_ref, out_ref):
    out_ref[...] = jnp.dot(jnp.sum(p_ref[...], axis=0), wh_ref[...],
                           preferred_element_type=jnp.float32)


def _sc_edge_kernel(acc_rows, d, a_dim, ng):
    rows_per_sub = acc_rows // NS

    mesh = plsc.VectorSubcoreMesh(core_axis_name="c", subcore_axis_name="s",
                                  num_cores=NC)

    @functools.partial(
        pl.kernel,
        out_type=pltpu.HBM((NC, acc_rows, d), jnp.float32),
        mesh=mesh,
        compiler_params=pltpu.CompilerParams(needs_layout_passes=False),
        scratch_types=[
            [pltpu.VMEM((G, 8), jnp.int32)] * 2,
            [pltpu.VMEM((G,), jnp.int32)] * 2,
            [pltpu.VMEM((G,), jnp.int32)] * 2,
            [pltpu.VMEM((G,), jnp.int32)] * 2,
            [pltpu.VMEM((G,), jnp.int32)] * 2,
            [pltpu.VMEM((G, d), jnp.float32)] * 2,
            [pltpu.VMEM((G, d), jnp.float32)] * 2,
            [pltpu.VMEM((G, d), jnp.float32)] * 2,
            [pltpu.VMEM((G, d), jnp.float32)] * 2,
            pltpu.VMEM((G,), jnp.float32),
            pltpu.VMEM((4, a_dim), jnp.float32),
            pltpu.VMEM((a_dim, 16), jnp.float32),
            pltpu.VMEM((16,), jnp.float32),
            pltpu.VMEM((16,), jnp.int32),
            pltpu.VMEM_SHARED((acc_rows, d), jnp.float32),
            pltpu.SemaphoreType.DMA,
            pltpu.SemaphoreType.DMA,
        ],
    )
    def sc_kernel(e8_hbm, hid_hbm, rela_hbm, aws_hbm, awr_hbm, combo_hbm,
                  w_hbm, wb_hbm, lf_hbm, out_hbm,
                  ev, sub_v, rel_v, obj_v, c_v, hs_v, hr_v, as_v, ar_v,
                  alpha_v, combo_v, w_v, wb_v, lf_v,
                  accum, sem, sem_e):
        c = lax.axis_index("c")
        s = lax.axis_index("s")
        wid = s * NC + c

        pltpu.sync_copy(combo_hbm, combo_v)
        pltpu.sync_copy(w_hbm, w_v)
        pltpu.sync_copy(wb_hbm, wb_v)
        pltpu.sync_copy(lf_hbm, lf_v)

        def zrow(r, carry):
            for k in range(d // 16):
                hs_v[0][r, pl.ds(16 * k, 16)] = jnp.zeros((16,), jnp.float32)
            return carry
        lax.fori_loop(0, G, zrow, 0)
        for j in range(rows_per_sub // G):
            pltpu.sync_copy(hs_v[0],
                            accum.at[pl.ds(s * rows_per_sub + j * G, G)])
        rem = rows_per_sub % G
        if rem:
            pltpu.sync_copy(
                hs_v[0].at[pl.ds(0, rem)],
                accum.at[pl.ds(s * rows_per_sub + rows_per_sub - rem, rem)])
        plsc.subcore_barrier()

        left_vec = lf_v[...]
        wb_vec = wb_v[...]

        def ebase(g):
            return (g * NW + wid) * G

        def extract_and_fire(h):
            for q in range(G // 16):
                evec = lax.iota(jnp.int32, 16) + q * 16
                head = plsc.load_gather(ev[h], [evec, jnp.full((16,), 1, jnp.int32)])
                rel = plsc.load_gather(ev[h], [evec, jnp.full((16,), 2, jnp.int32)])
                tail = plsc.load_gather(ev[h], [evec, jnp.full((16,), 3, jnp.int32)])
                sub = plsc.load_gather(ev[h], [evec, jnp.full((16,), 4, jnp.int32)])
                obj = plsc.load_gather(ev[h], [evec, jnp.full((16,), 5, jnp.int32)])
                cvec = (2 * (head >= left_vec).astype(jnp.int32)
                        + (tail >= left_vec).astype(jnp.int32))
                sub_v[h][pl.ds(q * 16, 16)] = sub
                rel_v[h][pl.ds(q * 16, 16)] = rel
                obj_v[h][pl.ds(q * 16, 16)] = obj
                c_v[h][pl.ds(q * 16, 16)] = cvec
            pltpu.async_copy(hid_hbm.at[sub_v[h]], hs_v[h], sem)
            pltpu.async_copy(rela_hbm.at[rel_v[h]], hr_v[h], sem)
            pltpu.async_copy(aws_hbm.at[sub_v[h]], as_v[h], sem)
            pltpu.async_copy(awr_hbm.at[rel_v[h]], ar_v[h], sem)

        def drain_gathers(h):
            pltpu.make_async_copy(hid_hbm.at[sub_v[h]], hs_v[h], sem).wait()
            pltpu.make_async_copy(rela_hbm.at[rel_v[h]], hr_v[h], sem).wait()
            pltpu.make_async_copy(aws_hbm.at[sub_v[h]], as_v[h], sem).wait()
            pltpu.make_async_copy(awr_hbm.at[rel_v[h]], ar_v[h], sem).wait()

        asum_base = a_dim

        def compute(h):
            @plsc.parallel_loop(0, G, unroll=4)
            def asum_row(e):
                csp = plsc.load_gather(c_v[h], [jnp.full((16,), 0, jnp.int32) + e])
                for kq in range(a_dim // 16):
                    cb = plsc.load_gather(
                        combo_v, [csp, lax.iota(jnp.int32, 16) + kq * 16])
                    as_v[h][e, pl.ds(asum_base + kq * 16, 16)] = (
                        as_v[h][e, pl.ds(kq * 16, 16)]
                        + ar_v[h][e, pl.ds(kq * 16, 16)] + cb)

            for q in range(G // 16):
                evec = lax.iota(jnp.int32, 16) + q * 16
                acc = jnp.zeros((16,), jnp.float32)
                for k in range(a_dim):
                    av = plsc.load_gather(
                        as_v[h], [evec, jnp.full((16,), asum_base + k, jnp.int32)])
                    acc = acc + jnp.maximum(av, 0.0) * w_v[k, :]
                x = acc + wb_vec
                alpha = 1.0 / (1.0 + jnp.exp(-x))
                plsc.store_scatter(alpha_v, [evec], alpha)

            @plsc.parallel_loop(0, G, unroll=4)
            def msg_row(e):
                a_e = plsc.load_gather(alpha_v, [jnp.full((16,), 0, jnp.int32) + e])
                for sl in range(d // 16):
                    hr_v[h][e, pl.ds(16 * sl, 16)] = a_e * (
                        hs_v[h][e, pl.ds(16 * sl, 16)]
                        + hr_v[h][e, pl.ds(16 * sl, 16)])

            pltpu.sync_copy(hr_v[h], accum.at[obj_v[h]], add=True)

        pltpu.sync_copy(e8_hbm.at[pl.ds(ebase(0), G)], ev[0])
        pltpu.async_copy(e8_hbm.at[pl.ds(ebase(1), G)], ev[1], sem_e)
        extract_and_fire(0)

        def pair(i, carry):
            for h in range(2):
                g = 2 * i + h
                o = 1 - h
                pltpu.make_async_copy(
                    e8_hbm.at[pl.ds(ebase(g + 1), G)], ev[o], sem_e).wait()
                extract_and_fire(o)
                pltpu.async_copy(
                    e8_hbm.at[pl.ds(ebase(g + 2), G)], ev[h], sem_e)
                drain_gathers(h)
                compute(h)
            return carry
        lax.fori_loop(0, ng // 2, pair, 0)

        pltpu.make_async_copy(
            e8_hbm.at[pl.ds(ebase(ng + 1), G)], ev[1], sem_e).wait()
        drain_gathers(0)

        plsc.subcore_barrier()
        for j in range(rows_per_sub // G):
            r0 = s * rows_per_sub + j * G
            pltpu.sync_copy(accum.at[pl.ds(r0, G)], out_hbm.at[c, pl.ds(r0, G)])
        if rows_per_sub % G:
            rem2 = rows_per_sub % G
            r0 = s * rows_per_sub + rows_per_sub - rem2
            pltpu.sync_copy(accum.at[pl.ds(r0, rem2)],
                            out_hbm.at[c, pl.ds(r0, rem2)])

    return sc_kernel


def kernel(hidden, edges, n_node, kgemb, left_num, rela_embed, Ws_attn,
           Wr_attn, Wkg_attn_W, Wkg_attn_b, w_alpha_W, w_alpha_b, W_h):
    n, d = hidden.shape
    a_dim = Ws_attn.shape[1]
    e = edges.shape[0]

    ng = -(-e // (NW * G))
    ng = ng + (ng % 2)
    acc_rows = -(-(n + 1) // (NS * 8)) * (NS * 8)
    blk = 1000 if n % 1000 == 0 else 8 * (n // 8)
    grid = n // blk

    e6 = jnp.pad(edges.astype(jnp.int32), ((0, 0), (0, 2)))
    dummy = jnp.broadcast_to(
        jnp.array([0, 0, 0, 0, 0, n, 0, 0], jnp.int32),
        (NW * (ng + 2) * G - e, 8))
    e8 = jnp.concatenate([e6, dummy], axis=0)
    w_bc = jnp.broadcast_to(
        w_alpha_W.reshape((a_dim, 1)).astype(jnp.float32), (a_dim, 16))
    wb16 = jnp.broadcast_to(w_alpha_b.astype(jnp.float32), (16,))
    lf16 = jnp.broadcast_to(jnp.asarray(left_num, jnp.int32), (16,))
    wkgb = Wkg_attn_b.reshape((1, a_dim)).astype(jnp.float32)

    aws, awr, combo4 = pl.pallas_call(
        _precompute_body,
        grid=(grid,),
        in_specs=[
            pl.BlockSpec((blk, d), lambda i: (i, 0)),
            pl.BlockSpec((blk, d), lambda i: (i, 0)),
            pl.BlockSpec((d, a_dim), lambda i: (0, 0)),
            pl.BlockSpec((d, a_dim), lambda i: (0, 0)),
            pl.BlockSpec((2, d), lambda i: (0, 0)),
            pl.BlockSpec((2 * d, a_dim), lambda i: (0, 0)),
            pl.BlockSpec((1, a_dim), lambda i: (0, 0)),
        ],
        out_specs=[
            pl.BlockSpec((blk, d), lambda i: (i, 0)),
            pl.BlockSpec((blk, d), lambda i: (i, 0)),
            pl.BlockSpec((4, a_dim), lambda i: (0, 0)),
        ],
        out_shape=[
            jax.ShapeDtypeStruct((n, d), jnp.float32),
            jax.ShapeDtypeStruct((n, d), jnp.float32),
            jax.ShapeDtypeStruct((4, a_dim), jnp.float32),
        ],
    )(hidden, rela_embed, Ws_attn, Wr_attn, kgemb, Wkg_attn_W, wkgb)

    parts = _sc_edge_kernel(acc_rows, d, a_dim, ng)(
        e8, hidden, rela_embed, aws, awr, combo4, w_bc, wb16, lf16)

    if n % 1000 == 0:
        blk2, grid2 = 1000, n // 1000
    else:
        blk2, grid2 = acc_rows // NS, NS
    out_pad = pl.pallas_call(
        _final_body,
        grid=(grid2,),
        in_specs=[
            pl.BlockSpec((NC, blk2, d), lambda i: (0, i, 0)),
            pl.BlockSpec((d, d), lambda i: (0, 0)),
        ],
        out_specs=pl.BlockSpec((blk2, d), lambda i: (i, 0)),
        out_shape=jax.ShapeDtypeStruct((blk2 * grid2, d), jnp.float32),
    )(parts, W_h)

    return out_pad[:n]
